# baseline (speedup 1.0000x reference)
import functools

import jax
import jax.numpy as jnp
from jax import lax
from jax.experimental import pallas as pl
from jax.experimental.pallas import tpu as pltpu
from jax.experimental.pallas import tpu_sc as plsc

MEM_ROWS = 100000
HID = 512
NLAB = 128
BATCH_N = 16384
MIX = 0.5

NCORE = 2
NSUB = 16
LANES = 16
NWORK = NCORE * NSUB
ROWS_PER_W = BATCH_N // NWORK
WIN = 128
CHUNK = 8192

_mesh = plsc.VectorSubcoreMesh(core_axis_name="c", subcore_axis_name="s")


JPT = BATCH_N // NSUB
VPT = JPT // LANES
MEM_PAD = 100096
ZSTRIPE = MEM_PAD // NSUB


@functools.partial(
    pl.kernel,
    mesh=_mesh,
    out_type=jax.ShapeDtypeStruct((BATCH_N,), jnp.int32),
    scratch_types=[
        pltpu.VMEM((MEM_ROWS,), jnp.int32),
        pltpu.VMEM((JPT,), jnp.int32),
        pltpu.VMEM((JPT,), jnp.int32),
        pltpu.VMEM((JPT,), jnp.int32),
        pltpu.VMEM((8, 128), jnp.int32),
        pltpu.VMEM((ZSTRIPE,), jnp.int32),
        pltpu.VMEM_SHARED((MEM_PAD,), jnp.int32),
        pltpu.VMEM_SHARED((MEM_PAD,), jnp.int32),
    ],
    compiler_params=pltpu.CompilerParams(needs_layout_passes=False),
)
def _winner_call(idx_hbm, zero_hbm, w_hbm, aux, ibuf, vbuf, bbuf, iw2, zbuf, bits, wj):
    c = lax.axis_index("c")
    s = lax.axis_index("s")
    lane = lax.iota(jnp.int32, LANES)

    @pl.when(c == 0)
    def _():
        pltpu.sync_copy(zero_hbm.at[pl.ds(s * ZSTRIPE, ZSTRIPE)], zbuf)
        pltpu.sync_copy(zbuf, bits.at[pl.ds(s * ZSTRIPE, ZSTRIPE)])
        pltpu.sync_copy(zbuf, wj.at[pl.ds(s * ZSTRIPE, ZSTRIPE)])

        base = s * JPT
        pltpu.sync_copy(idx_hbm.at[pl.ds(base, JPT)], ibuf)

        def vec_a(v, carry):
            x = ibuf[pl.ds(v * LANES, LANES)]
            j = lane + (base + v * LANES)
            plsc.store_scatter(aux, [x], j)
            g = plsc.load_gather(aux, [x])
            ndup = jnp.sum((g != j).astype(jnp.int32))

            @pl.when(ndup > 0)
            def _fix():
                for l in range(LANES):
                    plsc.store_scatter(aux, [x], j, mask=lane == l)

            return carry

        lax.fori_loop(0, VPT, vec_a, 0)

        mybit = jnp.left_shift(jnp.int32(1), s)

        def vec_b(v, carry):
            x = ibuf[pl.ds(v * LANES, LANES)]
            j = lane + (base + v * LANES)
            g = plsc.load_gather(aux, [x])
            vbuf[pl.ds(v * LANES, LANES)] = jnp.where(g == j, mybit, 0)
            return carry

        lax.fori_loop(0, VPT, vec_b, 0)
        for k in range(8):
            for m in range(8):
                iw2[k, pl.ds(m * LANES, LANES)] = ibuf[
                    pl.ds(k * 128 + m * LANES, LANES)
                ]
        plsc.subcore_barrier()
        for k in range(8):
            pltpu.sync_copy(
                vbuf.at[pl.ds(k * 128, 128)], bits.at[iw2.at[k]], add=True
            )
        plsc.subcore_barrier()
        pltpu.sync_copy(bits.at[ibuf], bbuf)
        shift = s + 1

        def vec_c(v, carry):
            j = lane + (base + v * LANES)
            winloc = vbuf[pl.ds(v * LANES, LANES)] != 0
            bv = bbuf[pl.ds(v * LANES, LANES)]
            wing = jnp.logical_and(winloc, jnp.right_shift(bv, shift) == 0)
            vbuf[pl.ds(v * LANES, LANES)] = jnp.where(wing, j, 0)
            return carry

        lax.fori_loop(0, VPT, vec_c, 0)
        for k in range(8):
            pltpu.sync_copy(
                vbuf.at[pl.ds(k * 128, 128)], wj.at[iw2.at[k]], add=True
            )
        plsc.subcore_barrier()
        pltpu.sync_copy(wj.at[ibuf], bbuf)
        pltpu.sync_copy(bbuf, w_hbm.at[pl.ds(base, JPT)])


WINA = 64
WINB = 128


def _gather_windows(base, tab_hbm, ind_hbm, out_hbm, win, nwin, ibuf, bufs, sems):
    gsems, osems = sems
    gdesc = [None, None]
    odesc = [None, None]
    for p in range(2):
        pltpu.sync_copy(ind_hbm.at[pl.ds(base + p * win, win)], ibuf.at[p])
        gdesc[p] = pltpu.async_copy(tab_hbm.at[ibuf.at[p]], bufs[p], gsems[p])
    for t in range(nwin):
        p = t % 2
        gdesc[p].wait()
        odesc[p] = pltpu.async_copy(
            bufs[p], out_hbm.at[pl.ds(base + t * win, win)], osems[p]
        )
        if t + 2 < nwin:
            pltpu.sync_copy(
                ind_hbm.at[pl.ds(base + (t + 2) * win, win)], ibuf.at[p]
            )
            odesc[p].wait()
            odesc[p] = None
            gdesc[p] = pltpu.async_copy(tab_hbm.at[ibuf.at[p]], bufs[p], gsems[p])
    for p in range(2):
        if odesc[p] is not None:
            odesc[p].wait()


@functools.partial(
    pl.kernel,
    mesh=_mesh,
    out_type=jax.ShapeDtypeStruct((BATCH_N, HID), jnp.float32),
    scratch_types=[
        pltpu.VMEM((2, WINA), jnp.int32),
        pltpu.VMEM((WINA, HID), jnp.float32),
        pltpu.VMEM((WINA, HID), jnp.float32),
        pltpu.SemaphoreType.DMA,
        pltpu.SemaphoreType.DMA,
        pltpu.SemaphoreType.DMA,
        pltpu.SemaphoreType.DMA,
    ],
)
def _old_gather_call(mem_hbm, idx_hbm, old_hbm, iwa, rows0, rows1,
                     sem0, sem1, osem0, osem1):
    c = lax.axis_index("c")
    s = lax.axis_index("s")
    base = (s * NCORE + c) * ROWS_PER_W
    _gather_windows(base, mem_hbm, idx_hbm, old_hbm, WINA, ROWS_PER_W // WINA,
                    iwa, (rows0, rows1), ((sem0, sem1), (osem0, osem1)))


@functools.partial(
    pl.kernel,
    mesh=_mesh,
    out_type=jax.ShapeDtypeStruct((BATCH_N, NLAB), jnp.float32),
    scratch_types=[
        pltpu.VMEM((2, WINB), jnp.int32),
        pltpu.VMEM((WINB, NLAB), jnp.float32),
        pltpu.VMEM((WINB, NLAB), jnp.float32),
        pltpu.SemaphoreType.DMA,
        pltpu.SemaphoreType.DMA,
        pltpu.SemaphoreType.DMA,
        pltpu.SemaphoreType.DMA,
    ],
)
def _gvp_gather_call(vp_hbm, w_hbm, gvp_hbm, iwb, rb0, rb1,
                     sem0, sem1, osem0, osem1):
    c = lax.axis_index("c")
    s = lax.axis_index("s")
    base = (s * NCORE + c) * ROWS_PER_W
    _gather_windows(base, vp_hbm, w_hbm, gvp_hbm, WINB, ROWS_PER_W // WINB,
                    iwb, (rb0, rb1), ((sem0, sem1), (osem0, osem1)))


_BM1 = 2048


def _mm1_body(x_ref, wv_ref, wt_ref, o_ref, p_scr):
    @pl.when(pl.program_id(0) == 0)
    def _():
        p_scr[...] = jnp.dot(
            wv_ref[...], MIX * wt_ref[...], preferred_element_type=jnp.float32
        )

    o_ref[...] = jnp.dot(x_ref[...], p_scr[...], preferred_element_type=jnp.float32)


def _mm2_body(a_ref, g_ref, w_ref, b_ref, o_ref):
    o_ref[...] = (
        jnp.dot(a_ref[...], MIX * w_ref[...], preferred_element_type=jnp.float32)
        + g_ref[...]
        + b_ref[...]
    )


def kernel(mem, idx, val, W_v, W_tag, b_tag):
    idx32 = idx.astype(jnp.int32)

    valp = pl.pallas_call(
        _mm1_body,
        grid=(BATCH_N // _BM1,),
        in_specs=[
            pl.BlockSpec((_BM1, HID), lambda i: (i, 0)),
            pl.BlockSpec((HID, HID), lambda i: (0, 0)),
            pl.BlockSpec((HID, NLAB), lambda i: (0, 0)),
        ],
        out_specs=pl.BlockSpec((_BM1, NLAB), lambda i: (i, 0)),
        out_shape=jax.ShapeDtypeStruct((BATCH_N, NLAB), jnp.float32),
        scratch_shapes=[pltpu.VMEM((HID, NLAB), jnp.float32)],
    )(val, W_v, W_tag)

    w = _winner_call(idx32, jnp.zeros((MEM_PAD,), jnp.int32))
    old = _old_gather_call(mem, idx32)
    gvp = _gvp_gather_call(valp, w)

    bias = jnp.reshape(b_tag, (1, NLAB))
    out = pl.pallas_call(
        _mm2_body,
        grid=(BATCH_N // _BM1,),
        in_specs=[
            pl.BlockSpec((_BM1, HID), lambda i: (i, 0)),
            pl.BlockSpec((_BM1, NLAB), lambda i: (i, 0)),
            pl.BlockSpec((HID, NLAB), lambda i: (0, 0)),
            pl.BlockSpec((1, NLAB), lambda i: (0, 0)),
        ],
        out_specs=pl.BlockSpec((_BM1, NLAB), lambda i: (i, 0)),
        out_shape=jax.ShapeDtypeStruct((BATCH_N, NLAB), jnp.float32),
    )(old, gvp, W_tag, bias)
    return out


# speedup vs baseline: 1.0475x; 1.0475x over previous
import functools

import jax
import jax.numpy as jnp
from jax import lax
from jax.experimental import pallas as pl
from jax.experimental.pallas import tpu as pltpu
from jax.experimental.pallas import tpu_sc as plsc

MEM_ROWS = 100000
HID = 512
NLAB = 128
BATCH_N = 16384
MIX = 0.5

NCORE = 2
NSUB = 16
LANES = 16
NWORK = NCORE * NSUB
ROWS_PER_W = BATCH_N // NWORK
WIN = 128
CHUNK = 8192

_mesh = plsc.VectorSubcoreMesh(core_axis_name="c", subcore_axis_name="s")


JPT = BATCH_N // NSUB
VPT = JPT // LANES
MEM_PAD = 100096
ZSTRIPE = MEM_PAD // NSUB


@functools.partial(
    pl.kernel,
    mesh=_mesh,
    out_type=jax.ShapeDtypeStruct((BATCH_N,), jnp.int32),
    scratch_types=[
        pltpu.VMEM((MEM_ROWS,), jnp.int32),
        pltpu.VMEM((JPT,), jnp.int32),
        pltpu.VMEM((JPT,), jnp.int32),
        pltpu.VMEM((JPT,), jnp.int32),
        pltpu.VMEM((8, 128), jnp.int32),
        pltpu.VMEM((ZSTRIPE,), jnp.int32),
        pltpu.VMEM_SHARED((MEM_PAD,), jnp.int32),
        pltpu.VMEM_SHARED((MEM_PAD,), jnp.int32),
    ],
    compiler_params=pltpu.CompilerParams(needs_layout_passes=False),
)
def _winner_call(idx_hbm, zero_hbm, w_hbm, aux, ibuf, vbuf, bbuf, iw2, zbuf, bits, wj):
    c = lax.axis_index("c")
    s = lax.axis_index("s")
    lane = lax.iota(jnp.int32, LANES)

    @pl.when(c == 0)
    def _():
        pltpu.sync_copy(zero_hbm.at[pl.ds(s * ZSTRIPE, ZSTRIPE)], zbuf)
        pltpu.sync_copy(zbuf, bits.at[pl.ds(s * ZSTRIPE, ZSTRIPE)])
        pltpu.sync_copy(zbuf, wj.at[pl.ds(s * ZSTRIPE, ZSTRIPE)])

        base = s * JPT
        pltpu.sync_copy(idx_hbm.at[pl.ds(base, JPT)], ibuf)

        def vec_a(v, carry):
            x = ibuf[pl.ds(v * LANES, LANES)]
            j = lane + (base + v * LANES)
            plsc.store_scatter(aux, [x], j)
            g = plsc.load_gather(aux, [x])
            ndup = jnp.sum((g != j).astype(jnp.int32))

            @pl.when(ndup > 0)
            def _fix():
                for l in range(LANES):
                    plsc.store_scatter(aux, [x], j, mask=lane == l)

            return carry

        lax.fori_loop(0, VPT, vec_a, 0)

        mybit = jnp.left_shift(jnp.int32(1), s)

        def vec_b(v, carry):
            x = ibuf[pl.ds(v * LANES, LANES)]
            j = lane + (base + v * LANES)
            g = plsc.load_gather(aux, [x])
            vbuf[pl.ds(v * LANES, LANES)] = jnp.where(g == j, mybit, 0)
            return carry

        lax.fori_loop(0, VPT, vec_b, 0)
        for k in range(8):
            for m in range(8):
                iw2[k, pl.ds(m * LANES, LANES)] = ibuf[
                    pl.ds(k * 128 + m * LANES, LANES)
                ]
        plsc.subcore_barrier()
        for k in range(8):
            pltpu.sync_copy(
                vbuf.at[pl.ds(k * 128, 128)], bits.at[iw2.at[k]], add=True
            )
        plsc.subcore_barrier()
        pltpu.sync_copy(bits.at[ibuf], bbuf)
        shift = s + 1

        def vec_c(v, carry):
            j = lane + (base + v * LANES)
            winloc = vbuf[pl.ds(v * LANES, LANES)] != 0
            bv = bbuf[pl.ds(v * LANES, LANES)]
            wing = jnp.logical_and(winloc, jnp.right_shift(bv, shift) == 0)
            vbuf[pl.ds(v * LANES, LANES)] = jnp.where(wing, j, 0)
            return carry

        lax.fori_loop(0, VPT, vec_c, 0)
        for k in range(8):
            pltpu.sync_copy(
                vbuf.at[pl.ds(k * 128, 128)], wj.at[iw2.at[k]], add=True
            )
        plsc.subcore_barrier()
        pltpu.sync_copy(wj.at[ibuf], bbuf)
        pltpu.sync_copy(bbuf, w_hbm.at[pl.ds(base, JPT)])


WINA = 64
WINB = 128


def _gather_windows(base, tab_hbm, ind_hbm, out_hbm, win, nwin, ibuf, bufs, sems):
    gsems, osems = sems
    gdesc = [None, None]
    odesc = [None, None]
    for p in range(2):
        pltpu.sync_copy(ind_hbm.at[pl.ds(base + p * win, win)], ibuf.at[p])
        gdesc[p] = pltpu.async_copy(tab_hbm.at[ibuf.at[p]], bufs[p], gsems[p])
    for t in range(nwin):
        p = t % 2
        gdesc[p].wait()
        odesc[p] = pltpu.async_copy(
            bufs[p], out_hbm.at[pl.ds(base + t * win, win)], osems[p]
        )
        if t + 2 < nwin:
            pltpu.sync_copy(
                ind_hbm.at[pl.ds(base + (t + 2) * win, win)], ibuf.at[p]
            )
            odesc[p].wait()
            odesc[p] = None
            gdesc[p] = pltpu.async_copy(tab_hbm.at[ibuf.at[p]], bufs[p], gsems[p])
    for p in range(2):
        if odesc[p] is not None:
            odesc[p].wait()


@functools.partial(
    pl.kernel,
    mesh=_mesh,
    out_type=(
        jax.ShapeDtypeStruct((BATCH_N, HID), jnp.float32),
        jax.ShapeDtypeStruct((BATCH_N, NLAB), jnp.float32),
    ),
    scratch_types=[
        pltpu.VMEM((2, WINA), jnp.int32),
        pltpu.VMEM((2, WINB), jnp.int32),
        pltpu.VMEM((WINA, HID), jnp.float32),
        pltpu.VMEM((WINA, HID), jnp.float32),
        pltpu.VMEM((WINB, NLAB), jnp.float32),
        pltpu.VMEM((WINB, NLAB), jnp.float32),
        pltpu.SemaphoreType.DMA,
        pltpu.SemaphoreType.DMA,
        pltpu.SemaphoreType.DMA,
        pltpu.SemaphoreType.DMA,
    ],
)
def _gather_call(mem_hbm, idx_hbm, vp_hbm, w_hbm, old_hbm, gvp_hbm,
                 iwa, iwb, rows0, rows1, rb0, rb1, sem0, sem1, osem0, osem1):
    c = lax.axis_index("c")
    s = lax.axis_index("s")
    base = (s * NCORE + c) * ROWS_PER_W
    _gather_windows(base, mem_hbm, idx_hbm, old_hbm, WINA, ROWS_PER_W // WINA,
                    iwa, (rows0, rows1), ((sem0, sem1), (osem0, osem1)))
    _gather_windows(base, vp_hbm, w_hbm, gvp_hbm, WINB, ROWS_PER_W // WINB,
                    iwb, (rb0, rb1), ((sem0, sem1), (osem0, osem1)))


_BM1 = 2048


def _mm1_body(x_ref, wv_ref, wt_ref, o_ref, p_scr):
    @pl.when(pl.program_id(0) == 0)
    def _():
        p_scr[...] = jnp.dot(
            wv_ref[...], MIX * wt_ref[...], preferred_element_type=jnp.float32
        )

    o_ref[...] = jnp.dot(x_ref[...], p_scr[...], preferred_element_type=jnp.float32)


def _mm2_body(a_ref, g_ref, w_ref, b_ref, o_ref):
    o_ref[...] = (
        jnp.dot(a_ref[...], MIX * w_ref[...], preferred_element_type=jnp.float32)
        + g_ref[...]
        + b_ref[...]
    )


def kernel(mem, idx, val, W_v, W_tag, b_tag):
    idx32 = idx.astype(jnp.int32)

    valp = pl.pallas_call(
        _mm1_body,
        grid=(BATCH_N // _BM1,),
        in_specs=[
            pl.BlockSpec((_BM1, HID), lambda i: (i, 0)),
            pl.BlockSpec((HID, HID), lambda i: (0, 0)),
            pl.BlockSpec((HID, NLAB), lambda i: (0, 0)),
        ],
        out_specs=pl.BlockSpec((_BM1, NLAB), lambda i: (i, 0)),
        out_shape=jax.ShapeDtypeStruct((BATCH_N, NLAB), jnp.float32),
        scratch_shapes=[pltpu.VMEM((HID, NLAB), jnp.float32)],
    )(val, W_v, W_tag)

    w = _winner_call(idx32, jnp.zeros((MEM_PAD,), jnp.int32))
    old, gvp = _gather_call(mem, idx32, valp, w)

    bias = jnp.reshape(b_tag, (1, NLAB))
    out = pl.pallas_call(
        _mm2_body,
        grid=(BATCH_N // _BM1,),
        in_specs=[
            pl.BlockSpec((_BM1, HID), lambda i: (i, 0)),
            pl.BlockSpec((_BM1, NLAB), lambda i: (i, 0)),
            pl.BlockSpec((HID, NLAB), lambda i: (0, 0)),
            pl.BlockSpec((1, NLAB), lambda i: (0, 0)),
        ],
        out_specs=pl.BlockSpec((_BM1, NLAB), lambda i: (i, 0)),
        out_shape=jax.ShapeDtypeStruct((BATCH_N, NLAB), jnp.float32),
    )(old, gvp, W_tag, bias)
    return out
